# baseline (speedup 1.0000x reference)
import functools
import math

import numpy as np

import jax
import jax.numpy as jnp
from jax import lax
from jax.experimental import pallas as pl
from jax.experimental.pallas import tpu as pltpu
from jax.experimental.pallas import tpu_sc as plsc

SHAPE = (256, 256)
OVERSAMP = 1.125
WIDTH = 3
OS = tuple(int(np.ceil(OVERSAMP * n)) for n in SHAPE)
BETA = float(np.pi * (((WIDTH / OVERSAMP) * (OVERSAMP - 0.5)) ** 2 - 0.8) ** 0.5)
NCOIL = 8
NSAMP = 262144

NC, NS = 2, 16
NTILE = NC * NS
SPT = NSAMP // NTILE
CH = 256
NCHUNK = SPT // CH
GRIDPTS = OS[0] * OS[1]


@functools.lru_cache(maxsize=1)
def _dft_mats():
    n = np.arange(SHAPE[0])
    k = np.arange(OS[0])
    a = np.sqrt(BETA ** 2 - (np.pi * WIDTH * (n - SHAPE[0] // 2) / OS[0]) ** 2)
    apod = a / np.sinh(a)
    ph = np.exp(-2j * np.pi * np.outer(k - OS[0] // 2, n - SHAPE[0] // 2)
                / OS[0]) / np.sqrt(OS[0])
    A = ph * apod[None, :]
    return (jnp.asarray(A.real, dtype=jnp.float32),
            jnp.asarray(A.imag, dtype=jnp.float32))


def _make_table(image_real, image_imag):
    ar, ai = _dft_mats()
    em1 = functools.partial(jnp.einsum, 'kn,cnm->ckm',
                            precision=lax.Precision.HIGHEST)
    em2 = functools.partial(jnp.einsum, 'ckm,jm->kjc',
                            precision=lax.Precision.HIGHEST)
    ur = em1(ar, image_real) - em1(ai, image_imag)
    ui = em1(ar, image_imag) + em1(ai, image_real)
    gr = em2(ur, ar) - em2(ui, ai)
    gi = em2(ur, ai) + em2(ui, ar)
    tab = jnp.concatenate([gr, gi], axis=-1).reshape(GRIDPTS, 16)
    tab16 = tab.astype(jnp.bfloat16).reshape(GRIDPTS, 8, 2)
    return lax.bitcast_convert_type(tab16, jnp.int32)


_KB_COEF = [float((BETA * BETA / 4.0) ** k
                  / (math.factorial(k) ** 2) / np.i0(BETA))
            for k in range(11)]


def _kb(x):
    s = x * (2.0 / WIDTH)
    u = jnp.clip(1.0 - s * s, 0.0, None)
    acc = jnp.full_like(u, _KB_COEF[10])
    for k in range(9, -1, -1):
        acc = acc * u + _KB_COEF[k]
    return acc


def _sc_body(cxy_hbm, table_hbm, out_hbm, coord_v, idx_v, w_v, rows_v, out_v,
             sem_c, sem_g, sem_out):
    wid = lax.axis_index("s") * NC + lax.axis_index("c")
    s_base = wid * SPT

    def stage_coords(k, o, sem=None):
        src = cxy_hbm.at[:, pl.ds(s_base + k * CH, CH)]
        if sem is None:
            pltpu.sync_copy(src, coord_v.at[o])
        else:
            pltpu.async_copy(src, coord_v.at[o], sem)

    def wait_coords(k, o):
        pltpu.make_async_copy(cxy_hbm.at[:, pl.ds(s_base + k * CH, CH)],
                              coord_v.at[o], sem_c).wait()

    def compute_taps(o):
        @plsc.parallel_loop(0, CH // 16, unroll=4)
        def g_body(g):
            l0 = g * 16
            sl = pl.ds(l0, 16)
            cx = coord_v[o, 0, sl] * (float(OS[0]) / float(SHAPE[0])) \
                + float(OS[0] // 2)
            cy = coord_v[o, 1, sl] * (float(OS[1]) / float(SHAPE[1])) \
                + float(OS[1] // 2)
            bxi = cx.astype(jnp.int32)
            byi = cy.astype(jnp.int32)
            fx = cx - bxi.astype(jnp.float32)
            fy = cy - byi.astype(jnp.float32)
            wx = [_kb(fx + 1.0), _kb(fx), _kb(fx - 1.0)]
            wy = [_kb(fy + 1.0), _kb(fy), _kb(fy - 1.0)]

            def wrap(p, n):
                p = jnp.where(p < 0, p + n, p)
                return jnp.where(p >= n, p - n, p)

            ix = [wrap(bxi + d, OS[0]) for d in (-1, 0, 1)]
            iy = [wrap(byi + d, OS[1]) for d in (-1, 0, 1)]
            for a in range(3):
                for bq in range(3):
                    j = a * 3 + bq
                    idx_v[o, j, sl] = ix[a] * OS[1] + iy[bq]
                    w_v[o, j, sl] = wx[a] * wy[bq]

    def fire_gathers(b):
        for j in range(9):
            pltpu.async_copy(table_hbm.at[idx_v.at[b, j]], rows_v.at[b, j],
                             sem_g)

    def drain_gathers(b):
        for j in range(9):
            pltpu.make_async_copy(table_hbm.at[idx_v.at[b, j]],
                                  rows_v.at[b, j], sem_g).wait()

    def compute(b):
        bvec = jnp.full((16,), b, dtype=jnp.int32)

        @plsc.parallel_loop(0, CH // 16, unroll=4)
        def g_body(g):
            l0 = g * 16
            lanes = l0 + lax.iota(jnp.int32, 16)
            wv = [w_v[b, j, pl.ds(l0, 16)] for j in range(9)]

            for cp in range(8):
                cvec = jnp.full((16,), cp, dtype=jnp.int32)

                def tap(j):
                    jvec = jnp.full((16,), j, dtype=jnp.int32)
                    packed = plsc.load_gather(rows_v, [bvec, jvec, lanes, cvec])
                    return plsc.unpack(plsc.bitcast(packed, jnp.bfloat16),
                                       format=plsc.PackFormat.INTERLEAVED)

                t = [tap(j) for j in range(9)]
                for h in range(2):
                    t01 = wv[0] * t[0][h] + wv[1] * t[1][h]
                    t23 = wv[2] * t[2][h] + wv[3] * t[3][h]
                    t45 = wv[4] * t[4][h] + wv[5] * t[5][h]
                    t67 = wv[6] * t[6][h] + wv[7] * t[7][h]
                    acc = ((t01 + t23) + (t45 + t67)) + wv[8] * t[8][h]
                    out_v[b, 2 * cp + h, pl.ds(l0, 16)] = acc

    stage_coords(0, 0)
    compute_taps(0)
    fire_gathers(0)
    stage_coords(1, 1)

    def pair_body(kp, _):
        for p in range(2):
            k = 2 * kp + p
            b = p
            o = 1 - p
            drain_gathers(b)

            @pl.when(k + 1 < NCHUNK)
            def _():
                @pl.when(k >= 1)
                def _():
                    wait_coords(k + 1, o)

                compute_taps(o)
                fire_gathers(o)

            @pl.when(k + 2 < NCHUNK)
            def _():
                stage_coords(k + 2, b, sem_c)

            compute(b)

            @pl.when(k >= 1)
            def _():
                pltpu.make_async_copy(
                    out_v.at[o],
                    out_hbm.at[:, pl.ds(s_base + (k - 1) * CH, CH)],
                    sem_out).wait()

            pltpu.async_copy(out_v.at[b],
                             out_hbm.at[:, pl.ds(s_base + k * CH, CH)],
                             sem_out)

        return 0

    lax.fori_loop(0, NCHUNK // 2, pair_body, 0)
    pltpu.make_async_copy(
        out_v.at[1],
        out_hbm.at[:, pl.ds(s_base + (NCHUNK - 1) * CH, CH)],
        sem_out).wait()


@functools.lru_cache(maxsize=1)
def _get_sc_interp():
    return pl.kernel(
        _sc_body,
        out_type=jax.ShapeDtypeStruct((16, NSAMP), jnp.float32),
        mesh=plsc.VectorSubcoreMesh(core_axis_name="c", subcore_axis_name="s",
                                    num_cores=NC, num_subcores=NS),
        compiler_params=pltpu.CompilerParams(needs_layout_passes=False,
                                             use_tc_tiling_on_sc=False),
        scratch_types=[
            pltpu.VMEM((2, 2, CH), jnp.float32),
            pltpu.VMEM((2, 9, CH), jnp.int32),
            pltpu.VMEM((2, 9, CH), jnp.float32),
            pltpu.VMEM((2, 9, CH, 8), jnp.int32),
            pltpu.VMEM((2, 16, CH), jnp.float32),
            pltpu.SemaphoreType.DMA,
            pltpu.SemaphoreType.DMA,
            pltpu.SemaphoreType.DMA,
        ],
    )


@jax.jit
def kernel(image_real, image_imag, coord):
    table = _make_table(image_real, image_imag)
    cxy = coord.T
    out = _get_sc_interp()(cxy, table)
    return out.reshape(2, NCOIL, NSAMP)


# speedup vs baseline: 1.0386x; 1.0386x over previous
import functools
import math

import numpy as np

import jax
import jax.numpy as jnp
from jax import lax
from jax.experimental import pallas as pl
from jax.experimental.pallas import tpu as pltpu
from jax.experimental.pallas import tpu_sc as plsc

SHAPE = (256, 256)
OVERSAMP = 1.125
WIDTH = 3
OS = tuple(int(np.ceil(OVERSAMP * n)) for n in SHAPE)
BETA = float(np.pi * (((WIDTH / OVERSAMP) * (OVERSAMP - 0.5)) ** 2 - 0.8) ** 0.5)
NCOIL = 8
NSAMP = 262144

NC, NS = 2, 16
NTILE = NC * NS
SPT = NSAMP // NTILE
CH = 256
NCHUNK = SPT // CH
GRIDPTS = OS[0] * OS[1]


@functools.lru_cache(maxsize=1)
def _dft_mats():
    n = np.arange(SHAPE[0])
    k = np.arange(OS[0])
    a = np.sqrt(BETA ** 2 - (np.pi * WIDTH * (n - SHAPE[0] // 2) / OS[0]) ** 2)
    apod = a / np.sinh(a)
    ph = np.exp(-2j * np.pi * np.outer(k - OS[0] // 2, n - SHAPE[0] // 2)
                / OS[0]) / np.sqrt(OS[0])
    A = ph * apod[None, :]
    return (jnp.asarray(A.real, dtype=jnp.float32),
            jnp.asarray(A.imag, dtype=jnp.float32))


def _make_table(image_real, image_imag):
    ar, ai = _dft_mats()
    em1 = functools.partial(jnp.einsum, 'kn,cnm->ckm',
                            precision=lax.Precision.HIGH)
    em2 = functools.partial(jnp.einsum, 'ckm,jm->kjc',
                            precision=lax.Precision.HIGH)
    ur = em1(ar, image_real) - em1(ai, image_imag)
    ui = em1(ar, image_imag) + em1(ai, image_real)
    gr = em2(ur, ar) - em2(ui, ai)
    gi = em2(ur, ai) + em2(ui, ar)
    tab = jnp.concatenate([gr, gi], axis=-1).reshape(GRIDPTS, 16)
    tab16 = tab.astype(jnp.bfloat16).reshape(GRIDPTS, 8, 2)
    return lax.bitcast_convert_type(tab16, jnp.int32)


_KB_COEF = [float((BETA * BETA / 4.0) ** k
                  / (math.factorial(k) ** 2) / np.i0(BETA))
            for k in range(11)]


def _kb(x):
    s = x * (2.0 / WIDTH)
    u = jnp.clip(1.0 - s * s, 0.0, None)
    acc = jnp.full_like(u, _KB_COEF[10])
    for k in range(9, -1, -1):
        acc = acc * u + _KB_COEF[k]
    return acc


def _sc_body(cxy_hbm, table_hbm, out_hbm, coord_v, idx_v, w_v, rows_v, out_v,
             sem_c, sem_g, sem_out):
    wid = lax.axis_index("s") * NC + lax.axis_index("c")
    s_base = wid * SPT

    def stage_coords(k, o, sem=None):
        src = cxy_hbm.at[:, pl.ds(s_base + k * CH, CH)]
        if sem is None:
            pltpu.sync_copy(src, coord_v.at[o])
        else:
            pltpu.async_copy(src, coord_v.at[o], sem)

    def wait_coords(k, o):
        pltpu.make_async_copy(cxy_hbm.at[:, pl.ds(s_base + k * CH, CH)],
                              coord_v.at[o], sem_c).wait()

    def compute_taps(o):
        @plsc.parallel_loop(0, CH // 16, unroll=2)
        def g_body(g):
            l0 = g * 16
            sl = pl.ds(l0, 16)
            cx = coord_v[o, 0, sl] * (float(OS[0]) / float(SHAPE[0])) \
                + float(OS[0] // 2)
            cy = coord_v[o, 1, sl] * (float(OS[1]) / float(SHAPE[1])) \
                + float(OS[1] // 2)
            bxi = cx.astype(jnp.int32)
            byi = cy.astype(jnp.int32)
            fx = cx - bxi.astype(jnp.float32)
            fy = cy - byi.astype(jnp.float32)
            wx = [_kb(fx + 1.0), _kb(fx), _kb(fx - 1.0)]
            wy = [_kb(fy + 1.0), _kb(fy), _kb(fy - 1.0)]

            def wrap(p, n):
                p = jnp.where(p < 0, p + n, p)
                return jnp.where(p >= n, p - n, p)

            ix = [wrap(bxi + d, OS[0]) for d in (-1, 0, 1)]
            iy = [wrap(byi + d, OS[1]) for d in (-1, 0, 1)]
            for a in range(3):
                for bq in range(3):
                    j = a * 3 + bq
                    idx_v[o, j, sl] = ix[a] * OS[1] + iy[bq]
                    w_v[o, j, sl] = wx[a] * wy[bq]

    def fire_gathers(b):
        for j in range(9):
            pltpu.async_copy(table_hbm.at[idx_v.at[b, j]], rows_v.at[b, j],
                             sem_g)

    def drain_gathers(b):
        for j in range(9):
            pltpu.make_async_copy(table_hbm.at[idx_v.at[b, j]],
                                  rows_v.at[b, j], sem_g).wait()

    def compute(b):
        bvec = jnp.full((16,), b, dtype=jnp.int32)

        @plsc.parallel_loop(0, CH // 16, unroll=2)
        def g_body(g):
            l0 = g * 16
            lanes = l0 + lax.iota(jnp.int32, 16)
            wv = [w_v[b, j, pl.ds(l0, 16)] for j in range(9)]

            for cp in range(8):
                cvec = jnp.full((16,), cp, dtype=jnp.int32)

                def tap(j):
                    jvec = jnp.full((16,), j, dtype=jnp.int32)
                    packed = plsc.load_gather(rows_v, [bvec, jvec, lanes, cvec])
                    return plsc.unpack(plsc.bitcast(packed, jnp.bfloat16),
                                       format=plsc.PackFormat.INTERLEAVED)

                t = [tap(j) for j in range(9)]
                for h in range(2):
                    t01 = wv[0] * t[0][h] + wv[1] * t[1][h]
                    t23 = wv[2] * t[2][h] + wv[3] * t[3][h]
                    t45 = wv[4] * t[4][h] + wv[5] * t[5][h]
                    t67 = wv[6] * t[6][h] + wv[7] * t[7][h]
                    acc = ((t01 + t23) + (t45 + t67)) + wv[8] * t[8][h]
                    out_v[b, 2 * cp + h, pl.ds(l0, 16)] = acc

    stage_coords(0, 0)
    compute_taps(0)
    fire_gathers(0)
    stage_coords(1, 1)

    def pair_body(kp, _):
        for p in range(2):
            k = 2 * kp + p
            b = p
            o = 1 - p
            drain_gathers(b)

            @pl.when(k + 1 < NCHUNK)
            def _():
                @pl.when(k >= 1)
                def _():
                    wait_coords(k + 1, o)

                compute_taps(o)
                fire_gathers(o)

            @pl.when(k + 2 < NCHUNK)
            def _():
                stage_coords(k + 2, b, sem_c)

            compute(b)

            @pl.when(k >= 1)
            def _():
                pltpu.make_async_copy(
                    out_v.at[o],
                    out_hbm.at[:, pl.ds(s_base + (k - 1) * CH, CH)],
                    sem_out).wait()

            pltpu.async_copy(out_v.at[b],
                             out_hbm.at[:, pl.ds(s_base + k * CH, CH)],
                             sem_out)

        return 0

    lax.fori_loop(0, NCHUNK // 2, pair_body, 0)
    pltpu.make_async_copy(
        out_v.at[1],
        out_hbm.at[:, pl.ds(s_base + (NCHUNK - 1) * CH, CH)],
        sem_out).wait()


@functools.lru_cache(maxsize=1)
def _get_sc_interp():
    return pl.kernel(
        _sc_body,
        out_type=jax.ShapeDtypeStruct((16, NSAMP), jnp.float32),
        mesh=plsc.VectorSubcoreMesh(core_axis_name="c", subcore_axis_name="s",
                                    num_cores=NC, num_subcores=NS),
        compiler_params=pltpu.CompilerParams(needs_layout_passes=False,
                                             use_tc_tiling_on_sc=False),
        scratch_types=[
            pltpu.VMEM((2, 2, CH), jnp.float32),
            pltpu.VMEM((2, 9, CH), jnp.int32),
            pltpu.VMEM((2, 9, CH), jnp.float32),
            pltpu.VMEM((2, 9, CH, 8), jnp.int32),
            pltpu.VMEM((2, 16, CH), jnp.float32),
            pltpu.SemaphoreType.DMA,
            pltpu.SemaphoreType.DMA,
            pltpu.SemaphoreType.DMA,
        ],
    )


@jax.jit
def kernel(image_real, image_imag, coord):
    table = _make_table(image_real, image_imag)
    cxy = coord.T
    out = _get_sc_interp()(cxy, table)
    return out.reshape(2, NCOIL, NSAMP)
